# baseline (speedup 1.0000x reference)
import functools

import jax
import jax.numpy as jnp
from jax.experimental import pallas as pl
from jax.experimental.pallas import tpu as pltpu

B, S, D, U, TOPK = 1, 512, 768, 8, 2
DU = D // U
BR = 64
BC = 128


def _proj_kernel(x_ref, f_ref, wr_ref, wq_ref, wk_ref, wv_ref,
                 qg_ref, kgt_ref, vg_ref, vf_ref, kr_ref, w8t_ref,
                 cnt_ref, cumt_ref, indt_ref):
    x = x_ref[...]
    freqs = f_ref[...]

    swr = jax.nn.sigmoid(wr_ref[...])
    logits = jnp.dot(x, swr.T, preferred_element_type=jnp.float32)
    probs = jax.nn.softmax(logits, axis=-1)

    tri_r = jax.lax.broadcasted_iota(jnp.int32, (U, U), 0)
    tri_c = jax.lax.broadcasted_iota(jnp.int32, (U, U), 1)
    tri = (tri_r <= tri_c).astype(jnp.float32)

    p1 = jnp.max(probs, axis=1, keepdims=True)
    m1 = (probs == p1)
    occ1 = jnp.dot(m1.astype(jnp.float32), tri, preferred_element_type=jnp.float32)
    mask1 = m1 & (occ1 == 1.0)
    probs2 = jnp.where(mask1, -1.0, probs)
    p2 = jnp.max(probs2, axis=1, keepdims=True)
    m2 = (probs2 == p2)
    occ2 = jnp.dot(m2.astype(jnp.float32), tri, preferred_element_type=jnp.float32)
    mask2 = m2 & (occ2 == 1.0)
    ind = (mask1 | mask2).astype(jnp.float32)
    psum = p1 + p2 + 1e-9
    w8 = (p1 / psum) * mask1.astype(jnp.float32) + (p2 / psum) * mask2.astype(jnp.float32)
    w8t_ref[...] = w8.T.reshape(U, 1, S)

    colu = jax.lax.broadcasted_iota(jnp.int32, (U, D), 1) // DU
    rowu = jax.lax.broadcasted_iota(jnp.int32, (U, D), 0)
    e_mat = (colu == rowu).astype(jnp.float32)
    indw = jnp.dot(ind, e_mat, preferred_element_type=jnp.float32)

    colg = jax.lax.broadcasted_iota(jnp.int32, (DU // 2, D), 1)
    rowg = jax.lax.broadcasted_iota(jnp.int32, (DU // 2, D), 0)
    g_mat = ((colg % DU) // 2 == rowg).astype(jnp.float32)
    ft = jnp.dot(freqs, g_mat, preferred_element_type=jnp.float32)

    swq = jax.nn.sigmoid(wq_ref[...])
    swk = jax.nn.sigmoid(wk_ref[...])
    swv = jax.nn.sigmoid(wv_ref[...])

    q = jnp.dot(x, swq.T, preferred_element_type=jnp.float32)
    qa = jax.nn.sigmoid(q * indw * ft)
    k = jnp.dot(x, swk.T, preferred_element_type=jnp.float32)
    kr = k * indw * ft
    kr_ref[...] = kr
    ka = jax.nn.sigmoid(kr)
    vf = jnp.dot(x, swv.T, preferred_element_type=jnp.float32) * indw
    vf_ref[...] = vf

    indt = ind.T
    indt_ref[...] = indt.reshape(U, 1, S)
    iota_s1 = jax.lax.broadcasted_iota(jnp.int32, (S, S), 0)
    iota_s2 = jax.lax.broadcasted_iota(jnp.int32, (S, S), 1)
    triu_s = (iota_s1 <= iota_s2).astype(jnp.float32)
    cumt = jnp.dot(indt, triu_s, preferred_element_type=jnp.float32)
    cumti = cumt.astype(jnp.int32)
    cumt_ref[...] = cumti.reshape(U, 1, S)
    cnt_ref[...] = cumti[:, S - 1:S]

    iota_p = jax.lax.broadcasted_iota(jnp.int32, (S, S), 0)
    for u in range(U):
        pmat = ((cumti[u:u + 1, :] == iota_p + 1)
                & (indt[u:u + 1, :] > 0.5)).astype(jnp.float32)
        qg_ref[u] = jnp.dot(pmat, qa[:, u * DU:(u + 1) * DU],
                            preferred_element_type=jnp.float32)
        kgt_ref[u] = jax.lax.dot_general(
            ka[:, u * DU:(u + 1) * DU], pmat,
            (((0,), (1,)), ((), ())),
            preferred_element_type=jnp.float32)
        vg_ref[u] = jnp.dot(pmat, vf[:, u * DU:(u + 1) * DU],
                            preferred_element_type=jnp.float32)


def _attn_scatter_kernel(cnt_ref, qg_ref, kgt_ref, vg_ref,
                         cumt_ref, indt_ref, w8t_ref, wout_ref,
                         fin_ref, og_ref):
    u = pl.program_id(0)
    n = cnt_ref[u]
    og_ref[...] = jnp.zeros((S, DU), jnp.float32)

    def row_body(rb, carry):
        base = rb * BR
        q = qg_ref[0, pl.ds(base, BR), :]
        qb = q.astype(jnp.bfloat16)
        acc_o = jnp.zeros((BR, DU), jnp.float32)
        acc_r = jnp.zeros((BR, 1), jnp.float32)
        for c in range(S // BC):
            def do(ao, ar, c=c):
                kblk = kgt_ref[0, :, c * BC:(c + 1) * BC].astype(jnp.bfloat16)
                vblk = vg_ref[0, c * BC:(c + 1) * BC, :]
                t3 = jax.nn.relu(qb[:, :, None] - kblk[None, :, :])
                ssum = jnp.sum(t3, axis=1, dtype=jnp.float32)
                tb = 1.0 - ssum * (1.0 / DU)
                cidx = c * BC + jax.lax.broadcasted_iota(jnp.int32, (BR, BC), 1)
                tb = jnp.where(cidx < n, tb, 0.0)
                return (ao + jnp.dot(tb, vblk, preferred_element_type=jnp.float32),
                        ar + jnp.sum(tb, axis=1, keepdims=True))
            acc_o, acc_r = jax.lax.cond(c * BC < n, do,
                                        lambda ao, ar: (ao, ar), acc_o, acc_r)
        a = jnp.sum(jax.nn.relu(q - 0.5), axis=1, keepdims=True)
        inact = 1.0 - a * (1.0 / DU)
        rs = acc_r + (jnp.float32(S) - n.astype(jnp.float32)) * inact
        og_ref[pl.ds(base, BR), :] = acc_o / (rs + 1e-9)
        return carry

    nrb = (n + BR - 1) // BR
    jax.lax.fori_loop(0, nrb, row_body, 0)

    iota_p = jax.lax.broadcasted_iota(jnp.int32, (S, S), 0)
    pmat_w = jnp.where((cumt_ref[0] == iota_p + 1) & (indt_ref[0] > 0.5),
                       w8t_ref[0], 0.0)
    od_scaled = jax.lax.dot_general(pmat_w, og_ref[...],
                                    (((0,), (0,)), ((), ())),
                                    preferred_element_type=jnp.float32)
    swo = jax.nn.sigmoid(wout_ref[0])
    part = jax.lax.dot_general(od_scaled, swo,
                               (((1,), (1,)), ((), ())),
                               preferred_element_type=jnp.float32)

    @pl.when(u == 0)
    def _():
        fin_ref[...] = part

    @pl.when(u != 0)
    def _():
        fin_ref[...] = fin_ref[...] + part


@functools.partial(jax.jit, static_argnames=("interpret",))
def _run(x, freqs_cis, Wr, Wq, Wk, Wv, Wout, interpret=False):
    x2 = x.reshape(S, D)
    wout_r = jnp.transpose(Wout.reshape(D, U, DU), (1, 0, 2))

    qg, kgt, vg, vf, kr, w8t, cnt, cumt, indt = pl.pallas_call(
        _proj_kernel,
        out_shape=[
            jax.ShapeDtypeStruct((U, S, DU), jnp.float32),
            jax.ShapeDtypeStruct((U, DU, S), jnp.float32),
            jax.ShapeDtypeStruct((U, S, DU), jnp.float32),
            jax.ShapeDtypeStruct((S, D), jnp.float32),
            jax.ShapeDtypeStruct((S, D), jnp.float32),
            jax.ShapeDtypeStruct((U, 1, S), jnp.float32),
            jax.ShapeDtypeStruct((U, 1), jnp.int32),
            jax.ShapeDtypeStruct((U, 1, S), jnp.int32),
            jax.ShapeDtypeStruct((U, 1, S), jnp.float32),
        ],
        interpret=interpret,
    )(x2, freqs_cis, Wr, Wq, Wk, Wv)

    final = pl.pallas_call(
        _attn_scatter_kernel,
        grid_spec=pltpu.PrefetchScalarGridSpec(
            num_scalar_prefetch=1,
            grid=(U,),
            in_specs=[
                pl.BlockSpec((1, S, DU), lambda u, cnt: (u, 0, 0)),
                pl.BlockSpec((1, DU, S), lambda u, cnt: (u, 0, 0)),
                pl.BlockSpec((1, S, DU), lambda u, cnt: (u, 0, 0)),
                pl.BlockSpec((1, 1, S), lambda u, cnt: (u, 0, 0)),
                pl.BlockSpec((1, 1, S), lambda u, cnt: (u, 0, 0)),
                pl.BlockSpec((1, 1, S), lambda u, cnt: (u, 0, 0)),
                pl.BlockSpec((1, D, DU), lambda u, cnt: (u, 0, 0)),
            ],
            out_specs=pl.BlockSpec((S, D), lambda u, cnt: (0, 0)),
            scratch_shapes=[pltpu.VMEM((S, DU), jnp.float32)],
        ),
        out_shape=jax.ShapeDtypeStruct((S, D), jnp.float32),
        interpret=interpret,
    )(cnt.reshape(U), qg, kgt, vg, cumt, indt, w8t, wout_r)

    return (final.reshape(B, S, D),
            kr.reshape(B, S, U, DU),
            vf.reshape(B, S, U, DU))


def kernel(x, freqs_cis, Wr, Wq, Wk, Wv, Wout):
    return _run(x, freqs_cis, Wr, Wq, Wk, Wv, Wout)


# speedup vs baseline: 1.0702x; 1.0702x over previous
import functools

import jax
import jax.numpy as jnp
from jax.experimental import pallas as pl
from jax.experimental.pallas import tpu as pltpu

B, S, D, U, TOPK = 1, 512, 768, 8, 2
DU = D // U
BR = 64
BC = 128


def _proj_kernel(x_ref, f_ref, wr_ref, wq_ref, wk_ref, wv_ref,
                 qg_ref, kgt_ref, vg_ref, vf_ref, kr_ref, w8t_ref,
                 cnt_ref, cumt_ref, indt_ref):
    x = x_ref[...]
    freqs = f_ref[...]

    swr = jax.nn.sigmoid(wr_ref[...])
    logits = jnp.dot(x, swr.T, preferred_element_type=jnp.float32)
    probs = jax.nn.softmax(logits, axis=-1)

    tri_r = jax.lax.broadcasted_iota(jnp.int32, (U, U), 0)
    tri_c = jax.lax.broadcasted_iota(jnp.int32, (U, U), 1)
    tri = (tri_r <= tri_c).astype(jnp.float32)

    p1 = jnp.max(probs, axis=1, keepdims=True)
    m1 = (probs == p1)
    occ1 = jnp.dot(m1.astype(jnp.float32), tri, preferred_element_type=jnp.float32)
    mask1 = m1 & (occ1 == 1.0)
    probs2 = jnp.where(mask1, -1.0, probs)
    p2 = jnp.max(probs2, axis=1, keepdims=True)
    m2 = (probs2 == p2)
    occ2 = jnp.dot(m2.astype(jnp.float32), tri, preferred_element_type=jnp.float32)
    mask2 = m2 & (occ2 == 1.0)
    ind = (mask1 | mask2).astype(jnp.float32)
    psum = p1 + p2 + 1e-9
    w8 = (p1 / psum) * mask1.astype(jnp.float32) + (p2 / psum) * mask2.astype(jnp.float32)
    w8t_ref[...] = w8.T.reshape(U, 1, S)

    colu = jax.lax.broadcasted_iota(jnp.int32, (U, D), 1) // DU
    rowu = jax.lax.broadcasted_iota(jnp.int32, (U, D), 0)
    e_mat = (colu == rowu).astype(jnp.float32)
    indw = jnp.dot(ind, e_mat, preferred_element_type=jnp.float32)

    colg = jax.lax.broadcasted_iota(jnp.int32, (DU // 2, D), 1)
    rowg = jax.lax.broadcasted_iota(jnp.int32, (DU // 2, D), 0)
    g_mat = ((colg % DU) // 2 == rowg).astype(jnp.float32)
    ft = jnp.dot(freqs, g_mat, preferred_element_type=jnp.float32)

    swq = jax.nn.sigmoid(wq_ref[...])
    swk = jax.nn.sigmoid(wk_ref[...])
    swv = jax.nn.sigmoid(wv_ref[...])

    q = jnp.dot(x, swq.T, preferred_element_type=jnp.float32)
    qa = jax.nn.sigmoid(q * indw * ft)
    k = jnp.dot(x, swk.T, preferred_element_type=jnp.float32)
    kr = k * indw * ft
    kr_ref[...] = kr
    ka = jax.nn.sigmoid(kr)
    vf = jnp.dot(x, swv.T, preferred_element_type=jnp.float32) * indw
    vf_ref[...] = vf

    indt = ind.T
    indt_ref[...] = indt.reshape(U, 1, S)
    iota_s1 = jax.lax.broadcasted_iota(jnp.int32, (S, S), 0)
    iota_s2 = jax.lax.broadcasted_iota(jnp.int32, (S, S), 1)
    triu_s = (iota_s1 <= iota_s2).astype(jnp.float32)
    cumt = jnp.dot(indt, triu_s, preferred_element_type=jnp.float32)
    cumti = cumt.astype(jnp.int32)
    cumt_ref[...] = cumti.reshape(U, 1, S)
    cnt_ref[...] = cumti[:, S - 1:S]

    iota_p = jax.lax.broadcasted_iota(jnp.int32, (S, S), 0)
    for u in range(U):
        pmat = ((cumti[u:u + 1, :] == iota_p + 1)
                & (indt[u:u + 1, :] > 0.5)).astype(jnp.float32)
        qg_ref[u] = jnp.dot(pmat, qa[:, u * DU:(u + 1) * DU],
                            preferred_element_type=jnp.float32)
        kgt_ref[u] = jax.lax.dot_general(
            ka[:, u * DU:(u + 1) * DU], pmat,
            (((0,), (1,)), ((), ())),
            preferred_element_type=jnp.float32)
        vg_ref[u] = jnp.dot(pmat, vf[:, u * DU:(u + 1) * DU],
                            preferred_element_type=jnp.float32)


def _attn_kernel(cnt_ref, qg_ref, kgt_ref, vg_ref, og_ref):
    u = pl.program_id(0)
    n = cnt_ref[u]
    og_ref[0] = jnp.zeros((S, DU), jnp.float32)

    def row_body(rb, carry):
        base = rb * BR
        q = qg_ref[0, pl.ds(base, BR), :]
        qb = q.astype(jnp.bfloat16)
        acc_o = jnp.zeros((BR, DU), jnp.float32)
        acc_r = jnp.zeros((BR, 1), jnp.float32)
        for c in range(S // BC):
            def do(ao, ar, c=c):
                kblk = kgt_ref[0, :, c * BC:(c + 1) * BC].astype(jnp.bfloat16)
                vblk = vg_ref[0, c * BC:(c + 1) * BC, :]
                t3 = jax.nn.relu(qb[:, :, None] - kblk[None, :, :])
                ssum = jnp.sum(t3, axis=1, dtype=jnp.float32)
                tb = 1.0 - ssum * (1.0 / DU)
                cidx = c * BC + jax.lax.broadcasted_iota(jnp.int32, (BR, BC), 1)
                tb = jnp.where(cidx < n, tb, 0.0)
                return (ao + jnp.dot(tb, vblk, preferred_element_type=jnp.float32),
                        ar + jnp.sum(tb, axis=1, keepdims=True))
            acc_o, acc_r = jax.lax.cond(c * BC < n, do,
                                        lambda ao, ar: (ao, ar), acc_o, acc_r)
        a = jnp.sum(jax.nn.relu(q - 0.5), axis=1, keepdims=True)
        inact = 1.0 - a * (1.0 / DU)
        rs = acc_r + (jnp.float32(S) - n.astype(jnp.float32)) * inact
        og_ref[0, pl.ds(base, BR), :] = acc_o / (rs + 1e-9)
        return carry

    nrb = (n + BR - 1) // BR
    jax.lax.fori_loop(0, nrb, row_body, 0)


def _outproj_kernel(og_ref, cumt_ref, indt_ref, w8t_ref, wout_ref, fin_ref):
    swo = jax.nn.sigmoid(wout_ref[...])
    iota_p = jax.lax.broadcasted_iota(jnp.int32, (S, S), 0)
    acc = jnp.zeros((S, D), dtype=jnp.float32)
    for u in range(U):
        pmat_w = jnp.where((cumt_ref[u] == iota_p + 1) & (indt_ref[u] > 0.5),
                           w8t_ref[u], 0.0)
        od_scaled = jax.lax.dot_general(pmat_w, og_ref[u],
                                        (((0,), (0,)), ((), ())),
                                        preferred_element_type=jnp.float32)
        acc = acc + jax.lax.dot_general(
            od_scaled, swo[:, u * DU:(u + 1) * DU],
            (((1,), (1,)), ((), ())),
            preferred_element_type=jnp.float32)
    fin_ref[...] = acc


@functools.partial(jax.jit, static_argnames=("interpret",))
def _run(x, freqs_cis, Wr, Wq, Wk, Wv, Wout, interpret=False):
    x2 = x.reshape(S, D)

    qg, kgt, vg, vf, kr, w8t, cnt, cumt, indt = pl.pallas_call(
        _proj_kernel,
        out_shape=[
            jax.ShapeDtypeStruct((U, S, DU), jnp.float32),
            jax.ShapeDtypeStruct((U, DU, S), jnp.float32),
            jax.ShapeDtypeStruct((U, S, DU), jnp.float32),
            jax.ShapeDtypeStruct((S, D), jnp.float32),
            jax.ShapeDtypeStruct((S, D), jnp.float32),
            jax.ShapeDtypeStruct((U, 1, S), jnp.float32),
            jax.ShapeDtypeStruct((U, 1), jnp.int32),
            jax.ShapeDtypeStruct((U, 1, S), jnp.int32),
            jax.ShapeDtypeStruct((U, 1, S), jnp.float32),
        ],
        interpret=interpret,
    )(x2, freqs_cis, Wr, Wq, Wk, Wv)

    og = pl.pallas_call(
        _attn_kernel,
        grid_spec=pltpu.PrefetchScalarGridSpec(
            num_scalar_prefetch=1,
            grid=(U,),
            in_specs=[
                pl.BlockSpec((1, S, DU), lambda u, cnt: (u, 0, 0)),
                pl.BlockSpec((1, DU, S), lambda u, cnt: (u, 0, 0)),
                pl.BlockSpec((1, S, DU), lambda u, cnt: (u, 0, 0)),
            ],
            out_specs=pl.BlockSpec((1, S, DU), lambda u, cnt: (u, 0, 0)),
        ),
        out_shape=jax.ShapeDtypeStruct((U, S, DU), jnp.float32),
        interpret=interpret,
    )(cnt.reshape(U), qg, kgt, vg)

    final = pl.pallas_call(
        _outproj_kernel,
        out_shape=jax.ShapeDtypeStruct((S, D), jnp.float32),
        interpret=interpret,
    )(og, cumt, indt, w8t, Wout)

    return (final.reshape(B, S, D),
            kr.reshape(B, S, U, DU),
            vf.reshape(B, S, U, DU))


def kernel(x, freqs_cis, Wr, Wq, Wk, Wv, Wout):
    return _run(x, freqs_cis, Wr, Wq, Wk, Wv, Wout)


# speedup vs baseline: 1.0957x; 1.0238x over previous
import functools

import jax
import jax.numpy as jnp
from jax.experimental import pallas as pl
from jax.experimental.pallas import tpu as pltpu

B, S, D, U, TOPK = 1, 512, 768, 8, 2
DU = D // U
BR = 64
BC = 128


def _proj_kernel(x_ref, f_ref, wr_ref, wq_ref, wk_ref, wv_ref,
                 qg_ref, kgt_ref, vg_ref, vf_ref, kr_ref, w8t_ref,
                 cnt_ref, cumt_ref, indt_ref):
    x = x_ref[...]
    freqs = f_ref[...]

    swr = jax.nn.sigmoid(wr_ref[...])
    logits = jnp.dot(x, swr.T, preferred_element_type=jnp.float32)
    probs = jax.nn.softmax(logits, axis=-1)

    tri_r = jax.lax.broadcasted_iota(jnp.int32, (U, U), 0)
    tri_c = jax.lax.broadcasted_iota(jnp.int32, (U, U), 1)
    tri = (tri_r <= tri_c).astype(jnp.float32)

    p1 = jnp.max(probs, axis=1, keepdims=True)
    m1 = (probs == p1)
    occ1 = jnp.dot(m1.astype(jnp.float32), tri, preferred_element_type=jnp.float32)
    mask1 = m1 & (occ1 == 1.0)
    probs2 = jnp.where(mask1, -1.0, probs)
    p2 = jnp.max(probs2, axis=1, keepdims=True)
    m2 = (probs2 == p2)
    occ2 = jnp.dot(m2.astype(jnp.float32), tri, preferred_element_type=jnp.float32)
    mask2 = m2 & (occ2 == 1.0)
    ind = (mask1 | mask2).astype(jnp.float32)
    psum = p1 + p2 + 1e-9
    w8 = (p1 / psum) * mask1.astype(jnp.float32) + (p2 / psum) * mask2.astype(jnp.float32)
    w8t_ref[...] = w8.T.reshape(U, 1, S)

    colu = jax.lax.broadcasted_iota(jnp.int32, (U, D), 1) // DU
    rowu = jax.lax.broadcasted_iota(jnp.int32, (U, D), 0)
    e_mat = (colu == rowu).astype(jnp.float32)
    indw = jnp.dot(ind, e_mat, preferred_element_type=jnp.float32)

    colg = jax.lax.broadcasted_iota(jnp.int32, (DU // 2, D), 1)
    rowg = jax.lax.broadcasted_iota(jnp.int32, (DU // 2, D), 0)
    g_mat = ((colg % DU) // 2 == rowg).astype(jnp.float32)
    ft = jnp.dot(freqs, g_mat, preferred_element_type=jnp.float32)

    swq = jax.nn.sigmoid(wq_ref[...])
    swk = jax.nn.sigmoid(wk_ref[...])
    swv = jax.nn.sigmoid(wv_ref[...])

    q = jnp.dot(x, swq.T, preferred_element_type=jnp.float32)
    qa = jax.nn.sigmoid(q * indw * ft)
    k = jnp.dot(x, swk.T, preferred_element_type=jnp.float32)
    kr = k * indw * ft
    kr_ref[...] = kr
    ka = jax.nn.sigmoid(kr)
    vf = jnp.dot(x, swv.T, preferred_element_type=jnp.float32) * indw
    vf_ref[...] = vf

    indt = ind.T
    indt_ref[...] = indt.reshape(U, 1, S)
    iota_s1 = jax.lax.broadcasted_iota(jnp.int32, (S, S), 0)
    iota_s2 = jax.lax.broadcasted_iota(jnp.int32, (S, S), 1)
    triu_s = (iota_s1 <= iota_s2).astype(jnp.float32)
    cumt = jnp.dot(indt, triu_s, preferred_element_type=jnp.float32)
    cumti = cumt.astype(jnp.int32)
    cumt_ref[...] = cumti.reshape(U, 1, S)
    cnt_ref[...] = cumti[:, S - 1:S]

    iota_p = jax.lax.broadcasted_iota(jnp.int32, (S, S), 0)
    for u in range(U):
        pmat = ((cumti[u:u + 1, :] == iota_p + 1)
                & (indt[u:u + 1, :] > 0.5)).astype(jnp.float32)
        qg_ref[u] = jnp.dot(pmat, qa[:, u * DU:(u + 1) * DU],
                            preferred_element_type=jnp.float32).astype(jnp.bfloat16)
        kgt_ref[u] = jax.lax.dot_general(
            ka[:, u * DU:(u + 1) * DU], pmat,
            (((0,), (1,)), ((), ())),
            preferred_element_type=jnp.float32).astype(jnp.bfloat16)
        vg_ref[u] = jnp.dot(pmat, vf[:, u * DU:(u + 1) * DU],
                            preferred_element_type=jnp.float32)


def _attn_kernel(cnt_ref, qg_ref, kgt_ref, vg_ref, og_ref):
    u = pl.program_id(0)
    n = cnt_ref[u]
    og_ref[0] = jnp.zeros((S, DU), jnp.float32)

    def row_body(rb, carry):
        base = rb * BR
        qb = qg_ref[0, pl.ds(base, BR), :]
        q = qb.astype(jnp.float32)
        acc_o = jnp.zeros((BR, DU), jnp.float32)
        acc_r = jnp.zeros((BR, 1), jnp.float32)
        for c in range(S // BC):
            def do(ao, ar, c=c):
                kblk = kgt_ref[0, :, c * BC:(c + 1) * BC]
                vblk = vg_ref[0, c * BC:(c + 1) * BC, :]
                t3 = jax.nn.relu(qb[:, :, None] - kblk[None, :, :])
                h1 = t3[:, :DU // 2, :] + t3[:, DU // 2:, :]
                h2 = h1[:, :DU // 4, :] + h1[:, DU // 4:, :]
                h3 = h2[:, :DU // 8, :] + h2[:, DU // 8:, :]
                ssum = jnp.sum(h3, axis=1, dtype=jnp.float32)
                tb = 1.0 - ssum * (1.0 / DU)
                cidx = c * BC + jax.lax.broadcasted_iota(jnp.int32, (BR, BC), 1)
                tb = jnp.where(cidx < n, tb, 0.0)
                return (ao + jnp.dot(tb, vblk, preferred_element_type=jnp.float32),
                        ar + jnp.sum(tb, axis=1, keepdims=True))
            acc_o, acc_r = jax.lax.cond(c * BC < n, do,
                                        lambda ao, ar: (ao, ar), acc_o, acc_r)
        a = jnp.sum(jax.nn.relu(q - 0.5), axis=1, keepdims=True)
        inact = 1.0 - a * (1.0 / DU)
        rs = acc_r + (jnp.float32(S) - n.astype(jnp.float32)) * inact
        og_ref[0, pl.ds(base, BR), :] = acc_o / (rs + 1e-9)
        return carry

    nrb = (n + BR - 1) // BR
    jax.lax.fori_loop(0, nrb, row_body, 0)


def _outproj_kernel(og_ref, cumt_ref, indt_ref, w8t_ref, wout_ref, fin_ref):
    swo = jax.nn.sigmoid(wout_ref[...])
    iota_p = jax.lax.broadcasted_iota(jnp.int32, (S, S), 0)
    acc = jnp.zeros((S, D), dtype=jnp.float32)
    for u in range(U):
        pmat_w = jnp.where((cumt_ref[u] == iota_p + 1) & (indt_ref[u] > 0.5),
                           w8t_ref[u], 0.0)
        od_scaled = jax.lax.dot_general(pmat_w, og_ref[u],
                                        (((0,), (0,)), ((), ())),
                                        preferred_element_type=jnp.float32)
        acc = acc + jax.lax.dot_general(
            od_scaled, swo[:, u * DU:(u + 1) * DU],
            (((1,), (1,)), ((), ())),
            preferred_element_type=jnp.float32)
    fin_ref[...] = acc


@functools.partial(jax.jit, static_argnames=("interpret",))
def _run(x, freqs_cis, Wr, Wq, Wk, Wv, Wout, interpret=False):
    x2 = x.reshape(S, D)

    qg, kgt, vg, vf, kr, w8t, cnt, cumt, indt = pl.pallas_call(
        _proj_kernel,
        out_shape=[
            jax.ShapeDtypeStruct((U, S, DU), jnp.bfloat16),
            jax.ShapeDtypeStruct((U, DU, S), jnp.bfloat16),
            jax.ShapeDtypeStruct((U, S, DU), jnp.float32),
            jax.ShapeDtypeStruct((S, D), jnp.float32),
            jax.ShapeDtypeStruct((S, D), jnp.float32),
            jax.ShapeDtypeStruct((U, 1, S), jnp.float32),
            jax.ShapeDtypeStruct((U, 1), jnp.int32),
            jax.ShapeDtypeStruct((U, 1, S), jnp.int32),
            jax.ShapeDtypeStruct((U, 1, S), jnp.float32),
        ],
        interpret=interpret,
    )(x2, freqs_cis, Wr, Wq, Wk, Wv)

    og = pl.pallas_call(
        _attn_kernel,
        grid_spec=pltpu.PrefetchScalarGridSpec(
            num_scalar_prefetch=1,
            grid=(U,),
            in_specs=[
                pl.BlockSpec((1, S, DU), lambda u, cnt: (u, 0, 0)),
                pl.BlockSpec((1, DU, S), lambda u, cnt: (u, 0, 0)),
                pl.BlockSpec((1, S, DU), lambda u, cnt: (u, 0, 0)),
            ],
            out_specs=pl.BlockSpec((1, S, DU), lambda u, cnt: (u, 0, 0)),
        ),
        out_shape=jax.ShapeDtypeStruct((U, S, DU), jnp.float32),
        interpret=interpret,
    )(cnt.reshape(U), qg, kgt, vg)

    final = pl.pallas_call(
        _outproj_kernel,
        out_shape=jax.ShapeDtypeStruct((S, D), jnp.float32),
        interpret=interpret,
    )(og, cumt, indt, w8t, Wout)

    return (final.reshape(B, S, D),
            kr.reshape(B, S, U, DU),
            vf.reshape(B, S, U, DU))


def kernel(x, freqs_cis, Wr, Wq, Wk, Wv, Wout):
    return _run(x, freqs_cis, Wr, Wq, Wk, Wv, Wout)
